# Initial kernel scaffold; baseline (speedup 1.0000x reference)
import jax
import jax.numpy as jnp
from jax.experimental import pallas as pl


def kernel(x, edge_index, batch, W1, b1, W2, b2, W3, b3, lin1_w, lin1_b, lin2_w, lin2_b):
    raise NotImplementedError("write your pallas kernel here")


# speedup vs baseline: 10.5406x; 10.5406x over previous
import functools

import jax
import jax.numpy as jnp
from jax import lax
from jax.experimental import pallas as pl
from jax.experimental.pallas import tpu as pltpu
from jax.experimental.pallas import tpu_sc as plsc

N = 10000
NP = 10240
E = 320000
D = 128
G = 64

NC = 2
NS = 16
NW = NC * NS
EW = E // NW
C = 80
NCHUNK = EW // C
RPW = NP // NS
ZR = 128

_mesh = plsc.VectorSubcoreMesh(core_axis_name="c", subcore_axis_name="s")


def _sc_body(do_gather):
    def body(hp_hbm, src_hbm, dst_hbm, zrows_hbm, ones_hbm, out,
             srcbuf, dstbuf, rows, zbuf, acc, sem):
        cid = lax.axis_index("c")
        sid = lax.axis_index("s")
        wid = sid * NC + cid

        pltpu.sync_copy(zrows_hbm, zbuf)
        for j in range(RPW // ZR):
            pltpu.sync_copy(zbuf, acc.at[pl.ds(sid * RPW + j * ZR, ZR)])
        if not do_gather:
            pltpu.sync_copy(ones_hbm, rows)
        plsc.subcore_barrier()

        base = wid * EW

        def _chunk(i, _):
            off = base + i * C
            pltpu.sync_copy(dst_hbm.at[pl.ds(off, C)], dstbuf)
            if do_gather:
                pltpu.sync_copy(src_hbm.at[pl.ds(off, C)], srcbuf)
                pltpu.async_copy(hp_hbm.at[srcbuf], rows, sem).wait()
            pltpu.sync_copy(rows, acc.at[dstbuf], add=True)
            return 0

        lax.fori_loop(0, NCHUNK, _chunk, 0)
        plsc.subcore_barrier()

        for j in range(RPW // ZR):
            pltpu.sync_copy(acc.at[pl.ds(sid * RPW + j * ZR, ZR)], zbuf)
            pltpu.sync_copy(zbuf,
                            out.at[pl.ds(cid * NP + sid * RPW + j * ZR, ZR)])
    return body


def _make_sc(do_gather):
    return functools.partial(
        pl.kernel,
        mesh=_mesh,
        out_type=jax.ShapeDtypeStruct((2 * NP, D), jnp.float32),
        scratch_types=[
            pltpu.VMEM((C,), jnp.int32),
            pltpu.VMEM((C,), jnp.int32),
            pltpu.VMEM((C, D), jnp.float32),
            pltpu.VMEM((ZR, D), jnp.float32),
            pltpu.VMEM_SHARED((NP, D), jnp.float32),
            pltpu.SemaphoreType.DMA,
        ],
    )(_sc_body(do_gather))


_sc_agg = _make_sc(True)
_sc_deg = _make_sc(False)


BLK = 1024
GRID = NP // BLK


def _dinv(deg0_ref, deg1_ref):
    deg = deg0_ref[:, 0:1] + deg1_ref[:, 0:1] + 1.0
    return lax.rsqrt(deg)


def _tc1_body(deg0_ref, deg1_ref, x_ref, w_ref, hp_ref):
    dinv = _dinv(deg0_ref, deg1_ref)
    h = jnp.dot(x_ref[...], w_ref[...], preferred_element_type=jnp.float32)
    hp_ref[...] = h * dinv


def _tc_mid_body(deg0_ref, deg1_ref, p0_ref, p1_ref, hp_ref, b_ref, w_ref,
                 out_ref):
    dinv = _dinv(deg0_ref, deg1_ref)
    conv = dinv * (p0_ref[...] + p1_ref[...] + hp_ref[...]) + b_ref[...]
    a = jnp.maximum(conv, 0.0)
    out_ref[...] = dinv * jnp.dot(a, w_ref[...],
                                  preferred_element_type=jnp.float32)


def _tc_head_body(deg0_ref, deg1_ref, p0_ref, p1_ref, hp_ref, b_ref,
                  batch_ref, l1w_ref, l1b_ref, l2w_ref, l2b_ref,
                  out_ref, sums, counts):
    i = pl.program_id(0)

    @pl.when(i == 0)
    def _():
        sums[...] = jnp.zeros_like(sums)
        counts[...] = jnp.zeros_like(counts)

    dinv = _dinv(deg0_ref, deg1_ref)
    conv = dinv * (p0_ref[...] + p1_ref[...] + hp_ref[...]) + b_ref[...]
    a = jnp.maximum(conv, 0.0)

    bt = batch_ref[0]
    oh = (jax.lax.broadcasted_iota(jnp.int32, (G, BLK), 0)
          == jnp.broadcast_to(bt, (G, BLK))).astype(jnp.float32)
    sums[...] += jnp.dot(oh, a, preferred_element_type=jnp.float32)
    counts[...] += jnp.broadcast_to(
        jnp.sum(oh, axis=1, keepdims=True), (G, D))

    @pl.when(i == GRID - 1)
    def _():
        g = sums[...] / jnp.maximum(counts[...], 1.0)
        z = jnp.maximum(
            jnp.dot(g, l1w_ref[...], preferred_element_type=jnp.float32)
            + l1b_ref[...], 0.0)
        out_ref[...] = (
            jnp.dot(z, l2w_ref[...], preferred_element_type=jnp.float32)
            + l2b_ref[...])


_row_spec = pl.BlockSpec((BLK, D), lambda i: (i, 0))
_w_spec = pl.BlockSpec((D, D), lambda i: (0, 0))
_b_spec = pl.BlockSpec((1, D), lambda i: (0, 0))

_tc1 = pl.pallas_call(
    _tc1_body,
    grid=(GRID,),
    in_specs=[_row_spec, _row_spec, _row_spec, _w_spec],
    out_specs=_row_spec,
    out_shape=jax.ShapeDtypeStruct((NP, D), jnp.float32),
)

_tc_mid = pl.pallas_call(
    _tc_mid_body,
    grid=(GRID,),
    in_specs=[_row_spec, _row_spec, _row_spec, _row_spec, _row_spec,
              _b_spec, _w_spec],
    out_specs=_row_spec,
    out_shape=jax.ShapeDtypeStruct((NP, D), jnp.float32),
)

_tc_head = pl.pallas_call(
    _tc_head_body,
    grid=(GRID,),
    in_specs=[_row_spec, _row_spec, _row_spec, _row_spec, _row_spec,
              _b_spec,
              pl.BlockSpec((1, 1, BLK), lambda i: (i, 0, 0)),
              _w_spec, _b_spec, _w_spec, _b_spec],
    out_specs=pl.BlockSpec((G, D), lambda i: (0, 0)),
    out_shape=jax.ShapeDtypeStruct((G, D), jnp.float32),
    scratch_shapes=[
        pltpu.VMEM((G, D), jnp.float32),
        pltpu.VMEM((G, D), jnp.float32),
    ],
)


def kernel(x, edge_index, batch, W1, b1, W2, b2, W3, b3,
           lin1_w, lin1_b, lin2_w, lin2_b):
    src = edge_index[0]
    dst = edge_index[1]
    xp = jnp.pad(x, ((0, NP - N), (0, 0)))
    batchp = jnp.pad(batch, (0, NP - N), constant_values=G)
    b1r = b1.reshape(1, D)
    b2r = b2.reshape(1, D)
    b3r = b3.reshape(1, D)
    l1br = lin1_b.reshape(1, D)
    l2br = lin2_b.reshape(1, D)
    batch3 = batchp.reshape(GRID, 1, BLK)

    ones_rows = jnp.ones((C, D), jnp.float32)
    zrows = jnp.zeros((ZR, D), jnp.float32)

    degc = _sc_deg(zrows, src, dst, zrows, ones_rows)
    deg0, deg1 = degc[:NP], degc[NP:]
    hp1 = _tc1(deg0, deg1, xp, W1)
    pc = _sc_agg(hp1, src, dst, zrows, ones_rows)
    p0, p1 = pc[:NP], pc[NP:]
    hp2 = _tc_mid(deg0, deg1, p0, p1, hp1, b1r, W2)
    pc = _sc_agg(hp2, src, dst, zrows, ones_rows)
    p0, p1 = pc[:NP], pc[NP:]
    hp3 = _tc_mid(deg0, deg1, p0, p1, hp2, b2r, W3)
    pc = _sc_agg(hp3, src, dst, zrows, ones_rows)
    p0, p1 = pc[:NP], pc[NP:]
    out = _tc_head(deg0, deg1, p0, p1, hp3, b3r, batch3,
                   lin1_w, l1br, lin2_w, l2br)
    return out


# speedup vs baseline: 19.4993x; 1.8499x over previous
import functools

import jax
import jax.numpy as jnp
from jax import lax
from jax.experimental import pallas as pl
from jax.experimental.pallas import tpu as pltpu
from jax.experimental.pallas import tpu_sc as plsc

N = 10000
NP = 10240
E = 320000
D = 128
G = 64

NC = 2
NS = 16
NW = NC * NS
EW = E // NW
C = 80
NCHUNK = EW // C
RPW = NP // NS

_mesh = plsc.VectorSubcoreMesh(core_axis_name="c", subcore_axis_name="s")


def _sc_body(do_gather):
    def body(hp_hbm, src_hbm, dst_hbm, zrows_hbm, ones_hbm, out,
             srcbufa, srcbufb, dstall, rowsa, rowsb, acc, sema, semb):
        cid = lax.axis_index("c")
        sid = lax.axis_index("s")
        wid = sid * NC + cid

        pltpu.sync_copy(zrows_hbm, rowsa)
        for j in range(RPW // C):
            pltpu.sync_copy(rowsa, acc.at[pl.ds(sid * RPW + j * C, C)])
        pltpu.sync_copy(dst_hbm.at[wid], dstall)
        if not do_gather:
            pltpu.sync_copy(ones_hbm, rowsa)
        plsc.subcore_barrier()

        if do_gather:
            pltpu.sync_copy(src_hbm.at[wid, 0], srcbufa)
            pltpu.async_copy(hp_hbm.at[srcbufa], rowsa, sema)

            def _pair(j, _):
                k = 2 * j
                pltpu.sync_copy(src_hbm.at[wid, k + 1], srcbufb)
                pltpu.async_copy(hp_hbm.at[srcbufb], rowsb, semb)
                pltpu.make_async_copy(hp_hbm.at[srcbufa], rowsa, sema).wait()
                pltpu.sync_copy(rowsa, acc.at[dstall.at[k]], add=True)
                pltpu.sync_copy(src_hbm.at[wid, k + 2], srcbufa)
                pltpu.async_copy(hp_hbm.at[srcbufa], rowsa, sema)
                pltpu.make_async_copy(hp_hbm.at[srcbufb], rowsb, semb).wait()
                pltpu.sync_copy(rowsb, acc.at[dstall.at[k + 1]], add=True)
                return 0

            lax.fori_loop(0, (NCHUNK - 1) // 2, _pair, 0)
            pltpu.make_async_copy(hp_hbm.at[srcbufa], rowsa, sema).wait()
            pltpu.sync_copy(rowsa, acc.at[dstall.at[NCHUNK - 1]], add=True)
        else:
            def _chunk(i, _):
                pltpu.sync_copy(rowsa, acc.at[dstall.at[i]], add=True)
                return 0

            lax.fori_loop(0, NCHUNK, _chunk, 0)
        plsc.subcore_barrier()

        for j in range(RPW // C):
            pltpu.sync_copy(acc.at[pl.ds(sid * RPW + j * C, C)], rowsa)
            pltpu.sync_copy(rowsa,
                            out.at[pl.ds(cid * NP + sid * RPW + j * C, C)])
    return body


def _make_sc(do_gather):
    return functools.partial(
        pl.kernel,
        mesh=_mesh,
        out_type=jax.ShapeDtypeStruct((2 * NP, D), jnp.float32),
        scratch_types=[
            pltpu.VMEM((C,), jnp.int32),
            pltpu.VMEM((C,), jnp.int32),
            pltpu.VMEM((NCHUNK, C), jnp.int32),
            pltpu.VMEM((C, D), jnp.float32),
            pltpu.VMEM((C, D), jnp.float32),
            pltpu.VMEM_SHARED((NP, D), jnp.float32),
            pltpu.SemaphoreType.DMA,
            pltpu.SemaphoreType.DMA,
        ],
    )(_sc_body(do_gather))


_sc_agg = _make_sc(True)
_sc_deg = _make_sc(False)


BLK = 1024
GRID = NP // BLK


def _dinv(deg0_ref, deg1_ref):
    deg = deg0_ref[:, 0:1] + deg1_ref[:, 0:1] + 1.0
    return lax.rsqrt(deg)


def _tc1_body(deg0_ref, deg1_ref, x_ref, w_ref, hp_ref):
    dinv = _dinv(deg0_ref, deg1_ref)
    h = jnp.dot(x_ref[...], w_ref[...], preferred_element_type=jnp.float32)
    hp_ref[...] = h * dinv


def _tc_mid_body(deg0_ref, deg1_ref, p0_ref, p1_ref, hp_ref, b_ref, w_ref,
                 out_ref):
    dinv = _dinv(deg0_ref, deg1_ref)
    conv = dinv * (p0_ref[...] + p1_ref[...] + hp_ref[...]) + b_ref[...]
    a = jnp.maximum(conv, 0.0)
    out_ref[...] = dinv * jnp.dot(a, w_ref[...],
                                  preferred_element_type=jnp.float32)


def _tc_head_body(deg0_ref, deg1_ref, p0_ref, p1_ref, hp_ref, b_ref,
                  batch_ref, l1w_ref, l1b_ref, l2w_ref, l2b_ref,
                  out_ref, sums, counts):
    i = pl.program_id(0)

    @pl.when(i == 0)
    def _():
        sums[...] = jnp.zeros_like(sums)
        counts[...] = jnp.zeros_like(counts)

    dinv = _dinv(deg0_ref, deg1_ref)
    conv = dinv * (p0_ref[...] + p1_ref[...] + hp_ref[...]) + b_ref[...]
    a = jnp.maximum(conv, 0.0)

    bt = batch_ref[0]
    oh = (jax.lax.broadcasted_iota(jnp.int32, (G, BLK), 0)
          == jnp.broadcast_to(bt, (G, BLK))).astype(jnp.float32)
    sums[...] += jnp.dot(oh, a, preferred_element_type=jnp.float32)
    counts[...] += jnp.broadcast_to(
        jnp.sum(oh, axis=1, keepdims=True), (G, D))

    @pl.when(i == GRID - 1)
    def _():
        g = sums[...] / jnp.maximum(counts[...], 1.0)
        z = jnp.maximum(
            jnp.dot(g, l1w_ref[...], preferred_element_type=jnp.float32)
            + l1b_ref[...], 0.0)
        out_ref[...] = (
            jnp.dot(z, l2w_ref[...], preferred_element_type=jnp.float32)
            + l2b_ref[...])


_row_spec = pl.BlockSpec((BLK, D), lambda i: (i, 0))
_w_spec = pl.BlockSpec((D, D), lambda i: (0, 0))
_b_spec = pl.BlockSpec((1, D), lambda i: (0, 0))

_tc1 = pl.pallas_call(
    _tc1_body,
    grid=(GRID,),
    in_specs=[_row_spec, _row_spec, _row_spec, _w_spec],
    out_specs=_row_spec,
    out_shape=jax.ShapeDtypeStruct((NP, D), jnp.float32),
)

_tc_mid = pl.pallas_call(
    _tc_mid_body,
    grid=(GRID,),
    in_specs=[_row_spec, _row_spec, _row_spec, _row_spec, _row_spec,
              _b_spec, _w_spec],
    out_specs=_row_spec,
    out_shape=jax.ShapeDtypeStruct((NP, D), jnp.float32),
)

_tc_head = pl.pallas_call(
    _tc_head_body,
    grid=(GRID,),
    in_specs=[_row_spec, _row_spec, _row_spec, _row_spec, _row_spec,
              _b_spec,
              pl.BlockSpec((1, 1, BLK), lambda i: (i, 0, 0)),
              _w_spec, _b_spec, _w_spec, _b_spec],
    out_specs=pl.BlockSpec((G, D), lambda i: (0, 0)),
    out_shape=jax.ShapeDtypeStruct((G, D), jnp.float32),
    scratch_shapes=[
        pltpu.VMEM((G, D), jnp.float32),
        pltpu.VMEM((G, D), jnp.float32),
    ],
)


def kernel(x, edge_index, batch, W1, b1, W2, b2, W3, b3,
           lin1_w, lin1_b, lin2_w, lin2_b):
    src = edge_index[0].reshape(NW, NCHUNK, C)
    dst = edge_index[1].reshape(NW, NCHUNK, C)
    xp = jnp.pad(x, ((0, NP - N), (0, 0)))
    batchp = jnp.pad(batch, (0, NP - N), constant_values=G)
    b1r = b1.reshape(1, D)
    b2r = b2.reshape(1, D)
    b3r = b3.reshape(1, D)
    l1br = lin1_b.reshape(1, D)
    l2br = lin2_b.reshape(1, D)
    batch3 = batchp.reshape(GRID, 1, BLK)

    ones_rows = jnp.ones((C, D), jnp.float32)
    zrows = jnp.zeros((C, D), jnp.float32)

    degc = _sc_deg(zrows, src, dst, zrows, ones_rows)
    deg0, deg1 = degc[:NP], degc[NP:]
    hp1 = _tc1(deg0, deg1, xp, W1)
    pc = _sc_agg(hp1, src, dst, zrows, ones_rows)
    p0, p1 = pc[:NP], pc[NP:]
    hp2 = _tc_mid(deg0, deg1, p0, p1, hp1, b1r, W2)
    pc = _sc_agg(hp2, src, dst, zrows, ones_rows)
    p0, p1 = pc[:NP], pc[NP:]
    hp3 = _tc_mid(deg0, deg1, p0, p1, hp2, b2r, W3)
    pc = _sc_agg(hp3, src, dst, zrows, ones_rows)
    p0, p1 = pc[:NP], pc[NP:]
    out = _tc_head(deg0, deg1, p0, p1, hp3, b3r, batch3,
                   lin1_w, l1br, lin2_w, l2br)
    return out
